# baseline (speedup 1.0000x reference)
import dataclasses
import functools

import jax
import jax.numpy as jnp
from jax.experimental import pallas as pl
from jax.experimental.pallas import tpu as pltpu
from jax.experimental.pallas import tpu_sc as plsc

NQ = 8
K = 8192
D = 256
B = 4096
BH = B // 2
BETA = 0.25

KB = 1024
NJ = K // KB
NCH = KB // 128
GATHER_WIN = 128


def _level_body(first, produce_wsq, r_ref, z_ref, cb_ref, *rest):
    if produce_wsq:
        loss_ref, idx_ref, rout_ref, lout_ref, wsqo_ref = rest
    else:
        wsqi_ref, loss_ref, idx_ref, rout_ref, lout_ref = rest

    if first:
        r = r_ref[...]
        lout_ref[0, 0] = loss_ref[0, 0]
    else:
        r_prev = r_ref[...]
        z = z_ref[...]
        t = z - r_prev
        qst = r_prev + t
        r = r_prev - qst
        lout_ref[0, 0] = loss_ref[0, 0] + jnp.sum(t * t)
    rout_ref[...] = r
    rb = (-2.0 * r).astype(jnp.bfloat16)
    zsq = jnp.sum(r * r, axis=1, keepdims=True)

    rmin = jnp.full((BH, 128), jnp.inf, dtype=jnp.float32)
    ridx = jnp.zeros((BH, 128), dtype=jnp.int32)
    for jb in range(NJ):
        wf = cb_ref[0, jb * KB:(jb + 1) * KB, :]
        if produce_wsq:
            wsq = jnp.sum(wf * wf, axis=1)
            wsqo_ref[0, jb * KB:(jb + 1) * KB] = wsq
        else:
            wsq = wsqi_ref[0, jb * KB:(jb + 1) * KB]
        mm2 = jax.lax.dot_general(
            rb, wf.astype(jnp.bfloat16),
            dimension_numbers=(((1,), (1,)), ((), ())),
            preferred_element_type=jnp.float32)
        d = (zsq + mm2) + wsq[None, :]
        for c in range(NCH):
            dc = d[:, c * 128:(c + 1) * 128]
            cid = jb * NCH + c
            upd = dc < rmin
            rmin = jnp.where(upd, dc, rmin)
            ridx = jnp.where(upd, cid, ridx)

    lane = jax.lax.broadcasted_iota(jnp.int32, (BH, 128), 1)
    jfull = ridx * 128 + lane
    vals_t = jnp.transpose(rmin, (1, 0))
    jf_t = jnp.transpose(jfull, (1, 0))
    colmin = jnp.min(vals_t, axis=0, keepdims=True)
    cand = jnp.where(vals_t == colmin, jf_t, jnp.int32(2**30))
    idx_ref[...] = jnp.min(cand, axis=0).reshape(1, BH)


def _level_call(q, first, r, z_prev, codebooks, loss, wsq_in=None):
    produce = wsq_in is None
    body = functools.partial(_level_body, first, produce)
    in_specs = [
        pl.BlockSpec((BH, D), lambda j: (0, 0)),
        pl.BlockSpec((BH, D), lambda j: (0, 0)),
        pl.BlockSpec((1, K, D), lambda j: (q, 0, 0)),
    ]
    args = [r, z_prev, codebooks]
    if not produce:
        in_specs.append(pl.BlockSpec((1, K), lambda j: (0, 0)))
        args.append(wsq_in)
    in_specs.append(pl.BlockSpec(memory_space=pltpu.SMEM))
    args.append(loss)
    out_specs = [
        pl.BlockSpec((1, BH), lambda j: (0, 0)),
        pl.BlockSpec((BH, D), lambda j: (0, 0)),
        pl.BlockSpec(memory_space=pltpu.SMEM),
    ]
    out_shape = [
        jax.ShapeDtypeStruct((1, BH), jnp.int32),
        jax.ShapeDtypeStruct((BH, D), jnp.float32),
        jax.ShapeDtypeStruct((1, 1), jnp.float32),
    ]
    if produce:
        out_specs.append(pl.BlockSpec((1, K), lambda j: (0, 0)))
        out_shape.append(jax.ShapeDtypeStruct((1, K), jnp.float32))
    return pl.pallas_call(
        body,
        grid=(1,),
        in_specs=in_specs,
        out_specs=out_specs,
        out_shape=out_shape,
    )(*args)


def _final_body(last, ze_ref, r_ref, z_ref, loss_ref, zq_ref, lo_ref):
    r_prev = r_ref[...]
    z = z_ref[...]
    t = z - r_prev
    qst = r_prev + t
    r = r_prev - qst
    zq_ref[...] = ze_ref[...] - r
    total = loss_ref[0, 0] + jnp.sum(t * t)
    if last:
        lo_ref[0, 0] = (1.0 + BETA) * total * (1.0 / (B * D))
    else:
        lo_ref[0, 0] = total


def _final_call(last, z_e, r, z_prev, loss):
    return pl.pallas_call(
        functools.partial(_final_body, last),
        in_specs=[
            pl.BlockSpec((BH, D), lambda: (0, 0)),
            pl.BlockSpec((BH, D), lambda: (0, 0)),
            pl.BlockSpec((BH, D), lambda: (0, 0)),
            pl.BlockSpec(memory_space=pltpu.SMEM),
        ],
        out_specs=[
            pl.BlockSpec((BH, D), lambda: (0, 0)),
            pl.BlockSpec(memory_space=pltpu.SMEM),
        ],
        out_shape=[
            jax.ShapeDtypeStruct((BH, D), jnp.float32),
            jax.ShapeDtypeStruct((1, 1), jnp.float32),
        ],
    )(z_e, r, z_prev, loss)


def _sc_gather(codebooks, idx, q):
    cp = pltpu.CompilerParams()
    if "needs_layout_passes" in pltpu.CompilerParams.__dataclass_fields__:
        cp = dataclasses.replace(cp, needs_layout_passes=False)

    @functools.partial(
        pl.kernel,
        out_type=jax.ShapeDtypeStruct((BH, D), jnp.float32),
        mesh=plsc.VectorSubcoreMesh(core_axis_name="core",
                                    subcore_axis_name="subcore"),
        compiler_params=cp,
    )
    def kern(cb_hbm, i_hbm, o_hbm):
        def body(i_vmem, o_vmem):
            pltpu.sync_copy(cb_hbm.at[q].at[i_vmem.at[0]], o_vmem)

        pltpu.emit_pipeline(
            body,
            grid=(BH // GATHER_WIN,),
            in_specs=[pl.BlockSpec((1, GATHER_WIN), lambda i: (0, i))],
            out_specs=[pl.BlockSpec((GATHER_WIN, D), lambda i: (i, 0))],
            core_axis_name=("core", "subcore"),
            dimension_semantics=(pltpu.PARALLEL,),
        )(i_hbm, o_hbm)

    return kern(codebooks, idx)


def kernel(z_e, codebooks):
    ze_a, ze_b = z_e[:BH], z_e[BH:]
    loss = jnp.zeros((1, 1), jnp.float32)
    r_a, r_b = ze_a, ze_b
    z_a, z_b = ze_a, ze_b
    for q in range(NQ):
        first = q == 0
        idx_a, r_a, loss, wsq = _level_call(q, first, r_a, z_a, codebooks, loss)
        z_a = _sc_gather(codebooks, idx_a, q)
        idx_b, r_b, loss = _level_call(q, first, r_b, z_b, codebooks, loss, wsq)
        z_b = _sc_gather(codebooks, idx_b, q)
    zq_a, loss = _final_call(False, ze_a, r_a, z_a, loss)
    zq_b, vq_loss = _final_call(True, ze_b, r_b, z_b, loss)
    return jnp.concatenate([zq_a, zq_b], axis=0), vq_loss.reshape(())


# speedup vs baseline: 1.2312x; 1.2312x over previous
import dataclasses
import functools

import jax
import jax.numpy as jnp
from jax.experimental import pallas as pl
from jax.experimental.pallas import tpu as pltpu
from jax.experimental.pallas import tpu_sc as plsc

NQ = 8
K = 8192
D = 256
B = 4096
BETA = 0.25

KB = 1024
NJ = K // KB
NCH = KB // 128
GATHER_WIN = 128


def _level_body(q, first, r_ref, z_ref, cb_ref, loss_ref, idx_ref, rout_ref,
                lout_ref, r_s, rb_s, zsq_s, rmin_s, ridx_s):
    j = pl.program_id(0)

    @pl.when(j == 0)
    def _update():
        if first:
            r = r_ref[...]
            lout_ref[0, 0] = loss_ref[0, 0]
        else:
            r_prev = r_ref[...]
            z = z_ref[...]
            t = z - r_prev
            qst = r_prev + t
            r = r_prev - qst
            lout_ref[0, 0] = loss_ref[0, 0] + jnp.sum(t * t)
        r_s[...] = r
        rout_ref[...] = r
        rb_s[...] = (-2.0 * r).astype(jnp.bfloat16)
        zsq_s[...] = jnp.sum(r * r, axis=1, keepdims=True)
        rmin_s[...] = jnp.full((B, 128), jnp.inf, dtype=jnp.float32)
        ridx_s[...] = jnp.zeros((B, 128), dtype=jnp.int32)

    wf = cb_ref[0]
    wsq = jnp.sum(wf * wf, axis=1)
    mm2 = jax.lax.dot_general(
        rb_s[...], wf.astype(jnp.bfloat16),
        dimension_numbers=(((1,), (1,)), ((), ())),
        preferred_element_type=jnp.float32)
    d = (zsq_s[...] + mm2) + wsq[None, :]

    rmin = rmin_s[...]
    ridx = ridx_s[...]
    for c in range(NCH):
        dc = d[:, c * 128:(c + 1) * 128]
        cid = j * NCH + c
        upd = dc < rmin
        rmin = jnp.minimum(dc, rmin)
        ridx = jnp.where(upd, cid, ridx)
    rmin_s[...] = rmin
    ridx_s[...] = ridx

    @pl.when(j == NJ - 1)
    def _finalize():
        vals = rmin_s[...]
        lane = jax.lax.broadcasted_iota(jnp.int32, (B, 128), 1)
        jfull = ridx_s[...] * 128 + lane
        vals_t = jnp.transpose(vals, (1, 0))
        jf_t = jnp.transpose(jfull, (1, 0))
        colmin = jnp.min(vals_t, axis=0, keepdims=True)
        cand = jnp.where(vals_t == colmin, jf_t, jnp.int32(2**30))
        idx_ref[...] = jnp.min(cand, axis=0).reshape(1, B)


def _level_call(q, first, r, z_prev, codebooks, loss):
    body = functools.partial(_level_body, q, first)
    return pl.pallas_call(
        body,
        grid=(NJ,),
        in_specs=[
            pl.BlockSpec((B, D), lambda j: (0, 0)),
            pl.BlockSpec((B, D), lambda j: (0, 0)),
            pl.BlockSpec((1, KB, D), lambda j: (q, j, 0)),
            pl.BlockSpec(memory_space=pltpu.SMEM),
        ],
        out_specs=[
            pl.BlockSpec((1, B), lambda j: (0, 0)),
            pl.BlockSpec((B, D), lambda j: (0, 0)),
            pl.BlockSpec(memory_space=pltpu.SMEM),
        ],
        out_shape=[
            jax.ShapeDtypeStruct((1, B), jnp.int32),
            jax.ShapeDtypeStruct((B, D), jnp.float32),
            jax.ShapeDtypeStruct((1, 1), jnp.float32),
        ],
        scratch_shapes=[
            pltpu.VMEM((B, D), jnp.float32),
            pltpu.VMEM((B, D), jnp.bfloat16),
            pltpu.VMEM((B, 1), jnp.float32),
            pltpu.VMEM((B, 128), jnp.float32),
            pltpu.VMEM((B, 128), jnp.int32),
        ],
    )(r, z_prev, codebooks, loss)


def _final_body(ze_ref, r_ref, z_ref, loss_ref, zq_ref, vl_ref):
    r_prev = r_ref[...]
    z = z_ref[...]
    t = z - r_prev
    qst = r_prev + t
    r = r_prev - qst
    zq_ref[...] = ze_ref[...] - r
    total = loss_ref[0, 0] + jnp.sum(t * t)
    vl_ref[0, 0] = (1.0 + BETA) * total * (1.0 / (B * D))


def _final_call(z_e, r, z_prev, loss):
    return pl.pallas_call(
        _final_body,
        in_specs=[
            pl.BlockSpec((B, D), lambda: (0, 0)),
            pl.BlockSpec((B, D), lambda: (0, 0)),
            pl.BlockSpec((B, D), lambda: (0, 0)),
            pl.BlockSpec(memory_space=pltpu.SMEM),
        ],
        out_specs=[
            pl.BlockSpec((B, D), lambda: (0, 0)),
            pl.BlockSpec(memory_space=pltpu.SMEM),
        ],
        out_shape=[
            jax.ShapeDtypeStruct((B, D), jnp.float32),
            jax.ShapeDtypeStruct((1, 1), jnp.float32),
        ],
    )(z_e, r, z_prev, loss)


def _sc_gather(codebooks, idx, q):
    cp = pltpu.CompilerParams()
    if "needs_layout_passes" in pltpu.CompilerParams.__dataclass_fields__:
        cp = dataclasses.replace(cp, needs_layout_passes=False)

    @functools.partial(
        pl.kernel,
        out_type=jax.ShapeDtypeStruct((B, D), jnp.float32),
        mesh=plsc.VectorSubcoreMesh(core_axis_name="core",
                                    subcore_axis_name="subcore"),
        compiler_params=cp,
    )
    def kern(cb_hbm, i_hbm, o_hbm):
        def body(i_vmem, o_vmem):
            pltpu.sync_copy(cb_hbm.at[q].at[i_vmem.at[0]], o_vmem)

        pltpu.emit_pipeline(
            body,
            grid=(B // GATHER_WIN,),
            in_specs=[pl.BlockSpec((1, GATHER_WIN), lambda i: (0, i))],
            out_specs=[pl.BlockSpec((GATHER_WIN, D), lambda i: (i, 0))],
            core_axis_name=("core", "subcore"),
            dimension_semantics=(pltpu.PARALLEL,),
        )(i_hbm, o_hbm)

    return kern(codebooks, idx)


def kernel(z_e, codebooks):
    loss = jnp.zeros((1, 1), jnp.float32)
    r = z_e
    z_prev = z_e
    for q in range(NQ):
        idx, r, loss = _level_call(q, q == 0, r, z_prev, codebooks, loss)
        z_prev = _sc_gather(codebooks, idx, q)
    z_q_sum, vq_loss = _final_call(z_e, r, z_prev, loss)
    return z_q_sum, vq_loss.reshape(())
